# baseline (speedup 1.0000x reference)
import functools

import jax
import jax.numpy as jnp
from jax import lax
from jax.experimental import pallas as pl
from jax.experimental.pallas import tpu as pltpu
from jax.experimental.pallas import tpu_sc as plsc

NUM_CLASSES = 80

_NC = 2
_NS = 16
_NW = _NC * _NS
_L = 16


def _sc_loc_kernel(n_total, n_anchors):
    per_w = n_total // _NW
    per_wa = n_anchors // _NW
    chunk = min(per_w, 16384)
    n_chunks = per_w // chunk
    chunk_a = per_wa // n_chunks
    assert chunk * n_chunks == per_w and chunk_a * n_chunks == per_wa
    assert chunk == chunk_a * 4

    mesh = plsc.VectorSubcoreMesh(
        core_axis_name="c", subcore_axis_name="s",
        num_cores=_NC, num_subcores=_NS)

    @functools.partial(
        pl.kernel,
        out_type=(
            jax.ShapeDtypeStruct((_NW, _L), jnp.float32),
            jax.ShapeDtypeStruct((_NW, _L), jnp.float32),
        ),
        mesh=mesh,
        scratch_types=[
            pltpu.VMEM((chunk,), jnp.float32),
            pltpu.VMEM((chunk,), jnp.float32),
            pltpu.VMEM((chunk_a,), jnp.int32),
            pltpu.VMEM((_L,), jnp.float32),
            pltpu.VMEM((_L,), jnp.float32),
        ],
    )
    def body(locp_hbm, loct_hbm, tgt_hbm, out_loc, out_np,
             locp_v, loct_v, tgt_v, accl_v, accn_v):
        wid = lax.axis_index("s") * _NC + lax.axis_index("c")
        base = wid * per_w
        abase = wid * per_wa
        sub_anchor = lax.shift_right_logical(lax.iota(jnp.int32, _L), 2)

        def chunk_body(ci, carry):
            accl, accn = carry
            off = pl.multiple_of(base + ci * chunk, 8)
            aoff = pl.multiple_of(abase + ci * chunk_a, 8)
            pltpu.sync_copy(locp_hbm.at[pl.ds(off, chunk)], locp_v)
            pltpu.sync_copy(loct_hbm.at[pl.ds(off, chunk)], loct_v)
            pltpu.sync_copy(tgt_hbm.at[pl.ds(aoff, chunk_a)], tgt_v)

            def vbody(ia, c2):
                al, an = c2
                tg16 = tgt_v[pl.ds(ia * _L, _L)]
                an = an + jnp.where(tg16 > 0, 1.0, 0.0)
                for k in range(4):
                    tgk = tg16.at[sub_anchor + (4 * k)].get(
                        mode="promise_in_bounds")
                    mfk = jnp.where(tgk > 0, 1.0, 0.0)
                    eoff = (ia * 4 + k) * _L
                    p = locp_v[pl.ds(eoff, _L)]
                    t = loct_v[pl.ds(eoff, _L)]
                    d = p - t
                    ad = jnp.abs(d)
                    elem = jnp.where(ad < 1.0, 0.5 * d * d, ad - 0.5)
                    al = al + mfk * elem
                return (al, an)

            return lax.fori_loop(0, chunk_a // _L, vbody, (accl, accn))

        zero = jnp.zeros((_L,), jnp.float32)
        accl, accn = lax.fori_loop(0, n_chunks, chunk_body, (zero, zero))
        accl_v[...] = accl
        accn_v[...] = accn
        pltpu.sync_copy(accl_v, out_loc.at[wid])
        pltpu.sync_copy(accn_v, out_np.at[wid])

    return body


_GROUP = 640
_ANCH_PER_ROW = _GROUP // NUM_CLASSES


def _tc_cls_body(tgt_ref, x_ref, out_ref):
    x = x_ref[...]
    tgt = tgt_ref[...].astype(jnp.float32)

    jj = lax.broadcasted_iota(jnp.int32, (_ANCH_PER_ROW, _GROUP), 1)
    aa = lax.broadcasted_iota(jnp.int32, (_ANCH_PER_ROW, _GROUP), 0)
    expand = (jj // NUM_CLASSES == aa).astype(jnp.float32)
    exp_tgt = jnp.dot(tgt, expand, preferred_element_type=jnp.float32)

    cls_id = lax.broadcasted_iota(jnp.int32, (1, _GROUP), 1) % NUM_CLASSES + 1
    t = exp_tgt == cls_id.astype(jnp.float32)

    s2 = jnp.where(t, -2.0, 2.0)
    z = x * s2 - 1.0
    sp = jnp.maximum(z, 0.0) + jnp.log1p(jnp.exp(-jnp.abs(z)))
    w2 = jnp.where(t, 0.125, 0.375)
    part = jnp.sum(w2 * sp)

    @pl.when(pl.program_id(0) == 0)
    def _():
        out_ref[0] = 0.0

    out_ref[0] += part


def _tc_cls(tgt2, x2, block_rows):
    n_rows = x2.shape[0]
    grid = n_rows // block_rows
    return pl.pallas_call(
        _tc_cls_body,
        grid=(grid,),
        in_specs=[
            pl.BlockSpec((block_rows, _ANCH_PER_ROW), lambda i: (i, 0)),
            pl.BlockSpec((block_rows, _GROUP), lambda i: (i, 0)),
        ],
        out_specs=pl.BlockSpec(memory_space=pltpu.SMEM),
        out_shape=jax.ShapeDtypeStruct((1,), jnp.float32),
    )(tgt2, x2)


def kernel(loc_preds, loc_targets, cls_preds, cls_targets):
    b, a, _ = loc_preds.shape
    n_anchors = b * a

    locp = loc_preds.reshape(-1)
    loct = loc_targets.reshape(-1)
    tgt_flat = cls_targets.reshape(-1)

    loc_parts, np_parts = _sc_loc_kernel(locp.shape[0], n_anchors)(
        locp, loct, tgt_flat)

    n_rows = n_anchors * NUM_CLASSES // _GROUP
    x2 = cls_preds.reshape(n_rows, _GROUP)
    tgt2 = cls_targets.reshape(n_rows, _ANCH_PER_ROW)
    cls_sum = _tc_cls(tgt2, x2, block_rows=512)

    loc_loss = jnp.sum(loc_parts)
    num_pos = jnp.sum(np_parts)
    return (loc_loss + cls_sum[0]) / num_pos


# speedup vs baseline: 6.3749x; 6.3749x over previous
import jax
import jax.numpy as jnp
from jax import lax
from jax.experimental import pallas as pl
from jax.experimental.pallas import tpu as pltpu

NUM_CLASSES = 80


def _body(tgt_ref, x_ref, lp_ref, lt_ref, out_ref):
    x = x_ref[0]
    tgt = tgt_ref[0]
    lp = lp_ref[0]
    lt = lt_ref[0]

    cls_id = lax.broadcasted_iota(jnp.int32, (NUM_CLASSES, 1), 0) + 1
    t = tgt == cls_id

    s2 = jnp.where(t, -2.0, 2.0)
    z = x * s2 - 1.0
    sp = jnp.maximum(z, 0.0) + jnp.log1p(jnp.exp(-jnp.abs(z)))
    w2 = jnp.where(t, 0.125, 0.375)
    cls_part = jnp.sum(w2 * sp)

    pos = tgt > 0
    np_part = jnp.sum(jnp.where(pos, 1.0, 0.0))

    d = lp - lt
    ad = jnp.abs(d)
    elem = jnp.where(ad < 1.0, 0.5 * d * d, ad - 0.5)
    loc_part = jnp.sum(jnp.where(pos, elem, 0.0))

    @pl.when((pl.program_id(0) == 0) & (pl.program_id(1) == 0))
    def _():
        out_ref[0] = 0.0
        out_ref[1] = 0.0
        out_ref[2] = 0.0

    out_ref[0] += cls_part
    out_ref[1] += loc_part
    out_ref[2] += np_part


def kernel(loc_preds, loc_targets, cls_preds, cls_targets):
    b, a, _ = loc_preds.shape

    xt = cls_preds.transpose(0, 2, 1)
    lpt = loc_preds.transpose(0, 2, 1)
    ltt = loc_targets.transpose(0, 2, 1)
    tgt3 = cls_targets.reshape(b, 1, a)

    lblk = 2048
    grid = (b, a // lblk)

    sums = pl.pallas_call(
        _body,
        grid=grid,
        in_specs=[
            pl.BlockSpec((1, 1, lblk), lambda i, j: (i, 0, j)),
            pl.BlockSpec((1, NUM_CLASSES, lblk), lambda i, j: (i, 0, j)),
            pl.BlockSpec((1, 4, lblk), lambda i, j: (i, 0, j)),
            pl.BlockSpec((1, 4, lblk), lambda i, j: (i, 0, j)),
        ],
        out_specs=pl.BlockSpec(memory_space=pltpu.SMEM),
        out_shape=jax.ShapeDtypeStruct((3,), jnp.float32),
    )(tgt3, xt, lpt, ltt)

    return (sums[0] + sums[1]) / sums[2]
